# baseline (speedup 1.0000x reference)
import functools

import jax
import jax.numpy as jnp
from jax import lax
from jax.experimental import pallas as pl
from jax.experimental.pallas import tpu as pltpu
from jax.experimental.pallas import tpu_sc as plsc

N = 128
M = 100000
B = 16384

CHUNK = 16384
GRID = (M + CHUNK - 1) // CHUNK

KCH = 128


def _stats_transpose_body(w_ref, wt_ref, logz_ref, m_ref, s_ref):
    i = pl.program_id(0)
    x = w_ref[...]
    xt = x.T
    row = i * CHUNK + lax.broadcasted_iota(jnp.int32, (CHUNK, N), 0)
    xt = jnp.where(row < M, xt, -jnp.inf)
    wt_ref[...] = xt

    @pl.when(i == 0)
    def _():
        m_ref[...] = jnp.full((1, N), -jnp.inf, jnp.float32)
        s_ref[...] = jnp.zeros((1, N), jnp.float32)

    cmax = jnp.max(xt, axis=0, keepdims=True)
    m_old = m_ref[...]
    m_new = jnp.maximum(m_old, cmax)
    s_new = (s_ref[...] * jnp.exp(m_old - m_new)
             + jnp.sum(jnp.exp(xt - m_new), axis=0, keepdims=True))
    m_ref[...] = m_new
    s_ref[...] = s_new

    @pl.when(i == GRID - 1)
    def _():
        logz_ref[...] = m_new + jnp.log(s_new)


def _make_sc_gather(nw, b_per_w):
    nch = b_per_w // KCH
    mesh = plsc.VectorSubcoreMesh(core_axis_name="c", subcore_axis_name="s")
    nc = plsc.get_sparse_core_info().num_cores

    @functools.partial(
        pl.kernel,
        mesh=mesh,
        out_type=jax.ShapeDtypeStruct((B, N), jnp.float32),
        scratch_types=[
            pltpu.VMEM((nch, KCH), jnp.int32),
            pltpu.VMEM((b_per_w, N), jnp.float32),
            pltpu.VMEM((N,), jnp.float32),
        ] + [pltpu.SemaphoreType.DMA] * 4,
    )
    def _gather(table_hbm, idx_hbm, logz_hbm, out_hbm,
                idx_v, rows_v, logz_v, s0, s1, s2, s3):
        sems = (s0, s1, s2, s3)
        wid = lax.axis_index("s") * nc + lax.axis_index("c")
        base = wid * b_per_w
        pltpu.sync_copy(idx_hbm.at[wid], idx_v)
        pltpu.sync_copy(logz_hbm, logz_v)
        copies = [
            pltpu.async_copy(table_hbm.at[idx_v.at[j]],
                             rows_v.at[pl.ds(j * KCH, KCH)], sems[j])
            for j in range(nch)
        ]

        lz = [logz_v[pl.ds(c * 16, 16)] for c in range(N // 16)]

        def sub_row(r, carry):
            for c in range(N // 16):
                rows_v[r, pl.ds(c * 16, 16)] = (
                    rows_v[r, pl.ds(c * 16, 16)] - lz[c])
            return carry

        for j in range(nch):
            copies[j].wait()
            lax.fori_loop(j * KCH, (j + 1) * KCH, sub_row, 0)

        pltpu.sync_copy(rows_v, out_hbm.at[pl.ds(base, b_per_w)])

    return _gather


def kernel(obervation_raw, W):
    info = plsc.get_sparse_core_info()
    nw = info.num_cores * info.num_subcores
    b_per_w = B // nw

    wt, logz = pl.pallas_call(
        _stats_transpose_body,
        grid=(GRID,),
        in_specs=[pl.BlockSpec((N, CHUNK), lambda i: (0, i))],
        out_specs=[
            pl.BlockSpec((CHUNK, N), lambda i: (i, 0)),
            pl.BlockSpec((1, N), lambda i: (0, 0)),
        ],
        out_shape=[
            jax.ShapeDtypeStruct((M, N), jnp.float32),
            jax.ShapeDtypeStruct((1, N), jnp.float32),
        ],
        scratch_shapes=[
            pltpu.VMEM((1, N), jnp.float32),
            pltpu.VMEM((1, N), jnp.float32),
        ],
    )(W)

    obs3 = obervation_raw.astype(jnp.int32).reshape(nw, b_per_w // KCH, KCH)
    out = _make_sc_gather(nw, b_per_w)(wt, obs3, logz.reshape(N))
    return out


# speedup vs baseline: 1.0004x; 1.0004x over previous
import functools

import jax
import jax.numpy as jnp
from jax import lax
from jax.experimental import pallas as pl
from jax.experimental.pallas import tpu as pltpu
from jax.experimental.pallas import tpu_sc as plsc

N = 128
M = 100000
B = 16384

CHUNK = 8192
GRID = (M + CHUNK - 1) // CHUNK

KCH = 128


def _stats_transpose_body(w_ref, wt_ref, logz_ref, m_ref, s_ref):
    i = pl.program_id(0)
    x = w_ref[...]
    xt = x.T
    row = i * CHUNK + lax.broadcasted_iota(jnp.int32, (CHUNK, N), 0)
    xt = jnp.where(row < M, xt, -jnp.inf)
    wt_ref[...] = xt

    @pl.when(i == 0)
    def _():
        m_ref[...] = jnp.full((1, N), -jnp.inf, jnp.float32)
        s_ref[...] = jnp.zeros((1, N), jnp.float32)

    cmax = jnp.max(xt, axis=0, keepdims=True)
    m_old = m_ref[...]
    m_new = jnp.maximum(m_old, cmax)
    s_new = (s_ref[...] * jnp.exp(m_old - m_new)
             + jnp.sum(jnp.exp(xt - m_new), axis=0, keepdims=True))
    m_ref[...] = m_new
    s_ref[...] = s_new

    @pl.when(i == GRID - 1)
    def _():
        logz_ref[...] = m_new + jnp.log(s_new)


def _make_sc_gather(nw, b_per_w):
    nch = b_per_w // KCH
    mesh = plsc.VectorSubcoreMesh(core_axis_name="c", subcore_axis_name="s")
    nc = plsc.get_sparse_core_info().num_cores

    @functools.partial(
        pl.kernel,
        mesh=mesh,
        out_type=jax.ShapeDtypeStruct((B, N), jnp.float32),
        scratch_types=[
            pltpu.VMEM((nch, KCH), jnp.int32),
            pltpu.VMEM((b_per_w, N), jnp.float32),
            pltpu.VMEM((N,), jnp.float32),
        ] + [pltpu.SemaphoreType.DMA] * 4,
    )
    def _gather(table_hbm, idx_hbm, logz_hbm, out_hbm,
                idx_v, rows_v, logz_v, s0, s1, s2, s3):
        sems = (s0, s1, s2, s3)
        wid = lax.axis_index("s") * nc + lax.axis_index("c")
        base = wid * b_per_w
        pltpu.sync_copy(idx_hbm.at[wid], idx_v)
        pltpu.sync_copy(logz_hbm, logz_v)
        copies = [
            pltpu.async_copy(table_hbm.at[idx_v.at[j]],
                             rows_v.at[pl.ds(j * KCH, KCH)], sems[j])
            for j in range(nch)
        ]

        lz = [logz_v[pl.ds(c * 16, 16)] for c in range(N // 16)]

        def sub_row(r, carry):
            for c in range(N // 16):
                rows_v[r, pl.ds(c * 16, 16)] = (
                    rows_v[r, pl.ds(c * 16, 16)] - lz[c])
            return carry

        for j in range(nch):
            copies[j].wait()
            lax.fori_loop(j * KCH, (j + 1) * KCH, sub_row, 0)

        pltpu.sync_copy(rows_v, out_hbm.at[pl.ds(base, b_per_w)])

    return _gather


def kernel(obervation_raw, W):
    info = plsc.get_sparse_core_info()
    nw = info.num_cores * info.num_subcores
    b_per_w = B // nw

    wt, logz = pl.pallas_call(
        _stats_transpose_body,
        grid=(GRID,),
        in_specs=[pl.BlockSpec((N, CHUNK), lambda i: (0, i))],
        out_specs=[
            pl.BlockSpec((CHUNK, N), lambda i: (i, 0)),
            pl.BlockSpec((1, N), lambda i: (0, 0)),
        ],
        out_shape=[
            jax.ShapeDtypeStruct((M, N), jnp.float32),
            jax.ShapeDtypeStruct((1, N), jnp.float32),
        ],
        scratch_shapes=[
            pltpu.VMEM((1, N), jnp.float32),
            pltpu.VMEM((1, N), jnp.float32),
        ],
    )(W)

    obs3 = obervation_raw.astype(jnp.int32).reshape(nw, b_per_w // KCH, KCH)
    out = _make_sc_gather(nw, b_per_w)(wt, obs3, logz.reshape(N))
    return out


# speedup vs baseline: 1.0266x; 1.0263x over previous
import functools

import jax
import jax.numpy as jnp
from jax import lax
from jax.experimental import pallas as pl
from jax.experimental.pallas import tpu as pltpu
from jax.experimental.pallas import tpu_sc as plsc

N = 128
M = 100000
B = 16384

CHUNK = 8192
GRID = (M + CHUNK - 1) // CHUNK

KCH = 128


def _stats_transpose_body(w_ref, wt_ref, logz_ref, m_ref, s_ref):
    i = pl.program_id(0)
    x = w_ref[...]
    xt = x.T
    row = i * CHUNK + lax.broadcasted_iota(jnp.int32, (CHUNK, N), 0)
    xt = jnp.where(row < M, xt, -jnp.inf)
    wt_ref[...] = xt

    @pl.when(i == 0)
    def _():
        m_ref[...] = jnp.full((1, N), -jnp.inf, jnp.float32)
        s_ref[...] = jnp.zeros((1, N), jnp.float32)

    cmax = jnp.max(xt, axis=0, keepdims=True)
    m_old = m_ref[...]
    m_new = jnp.maximum(m_old, cmax)
    s_new = (s_ref[...] * jnp.exp(m_old - m_new)
             + jnp.sum(jnp.exp(xt - m_new), axis=0, keepdims=True))
    m_ref[...] = m_new
    s_ref[...] = s_new

    @pl.when(i == GRID - 1)
    def _():
        logz_ref[...] = m_new + jnp.log(s_new)


def _make_sc_gather(nw, b_per_w):
    nch = b_per_w // KCH
    mesh = plsc.VectorSubcoreMesh(core_axis_name="c", subcore_axis_name="s")
    nc = plsc.get_sparse_core_info().num_cores

    @functools.partial(
        pl.kernel,
        mesh=mesh,
        out_type=jax.ShapeDtypeStruct((B, N), jnp.float32),
        scratch_types=[
            pltpu.VMEM((nch, KCH), jnp.int32),
            pltpu.VMEM((b_per_w, N), jnp.float32),
            pltpu.VMEM((1, N), jnp.float32),
        ] + [pltpu.SemaphoreType.DMA] * 4,
    )
    def _gather(table_hbm, idx_hbm, logz_hbm, out_hbm,
                idx_v, rows_v, logz_v, s0, s1, s2, s3):
        sems = (s0, s1, s2, s3)
        wid = lax.axis_index("s") * nc + lax.axis_index("c")
        base = wid * b_per_w
        pltpu.sync_copy(idx_hbm.at[wid], idx_v)
        pltpu.sync_copy(logz_hbm, logz_v)
        copies = [
            pltpu.async_copy(table_hbm.at[idx_v.at[j]],
                             rows_v.at[pl.ds(j * KCH, KCH)], sems[j])
            for j in range(nch)
        ]

        lz = [logz_v[0, pl.ds(c * 16, 16)] for c in range(N // 16)]

        def sub_row(r, carry):
            for c in range(N // 16):
                rows_v[r, pl.ds(c * 16, 16)] = (
                    rows_v[r, pl.ds(c * 16, 16)] - lz[c])
            return carry

        for j in range(nch):
            copies[j].wait()
            lax.fori_loop(j * KCH, (j + 1) * KCH, sub_row, 0)

        pltpu.sync_copy(rows_v, out_hbm.at[pl.ds(base, b_per_w)])

    return _gather


def kernel(obervation_raw, W):
    info = plsc.get_sparse_core_info()
    nw = info.num_cores * info.num_subcores
    b_per_w = B // nw

    wt, logz = pl.pallas_call(
        _stats_transpose_body,
        grid=(GRID,),
        in_specs=[pl.BlockSpec((N, CHUNK), lambda i: (0, i))],
        out_specs=[
            pl.BlockSpec((CHUNK, N), lambda i: (i, 0)),
            pl.BlockSpec((1, N), lambda i: (0, 0)),
        ],
        out_shape=[
            jax.ShapeDtypeStruct((M, N), jnp.float32),
            jax.ShapeDtypeStruct((1, N), jnp.float32),
        ],
        scratch_shapes=[
            pltpu.VMEM((1, N), jnp.float32),
            pltpu.VMEM((1, N), jnp.float32),
        ],
    )(W)

    obs3 = obervation_raw.astype(jnp.int32).reshape(nw, b_per_w // KCH, KCH)
    out = _make_sc_gather(nw, b_per_w)(wt, obs3, logz)
    return out
